# Initial kernel scaffold; baseline (speedup 1.0000x reference)
import jax
import jax.numpy as jnp
from jax.experimental import pallas as pl


def kernel(sequence, token_table, pe_weight):
    raise NotImplementedError("write your pallas kernel here")


# speedup vs baseline: 3.4920x; 3.4920x over previous
import functools

import jax
import jax.numpy as jnp
from jax import lax
from jax.experimental import pallas as pl
from jax.experimental.pallas import tpu as pltpu
from jax.experimental.pallas import tpu_sc as plsc

VOCAB = 100000
EMBED = 64
MAX_LEN = 200
BATCH = 4096

NUM_CORES = 2
NUM_SUBCORES = 16
NUM_WORKERS = NUM_CORES * NUM_SUBCORES
SEQ_PER_W = BATCH // NUM_WORKERS
CH = 2
N_CHUNKS = SEQ_PER_W // CH
ROWS = CH * MAX_LEN
LANES = 16
COLS = EMBED // LANES


def _body(seq_hbm, table_hbm, pe_hbm, out_hbm, pe_v, idx_v, rows_v, sem, osem):
    wid = lax.axis_index("s") * NUM_CORES + lax.axis_index("c")
    base = wid * SEQ_PER_W * MAX_LEN

    pltpu.sync_copy(pe_hbm, pe_v)

    def chunk(i, _):
        off = base + i * ROWS
        pltpu.sync_copy(seq_hbm.at[pl.ds(off, ROWS)], idx_v)
        pltpu.async_copy(table_hbm.at[idx_v], rows_v, sem).wait()

        def add_row(r, _):
            for c in range(COLS):
                pe_reg = pe_v[r, pl.ds(c * LANES, LANES)]
                for s in range(CH):
                    plsc.addupdate(
                        rows_v.at[s * MAX_LEN + r, pl.ds(c * LANES, LANES)],
                        pe_reg,
                    )
            return 0

        lax.fori_loop(0, MAX_LEN, add_row, 0, unroll=2)
        pltpu.async_copy(rows_v, out_hbm.at[pl.ds(off, ROWS)], osem).wait()
        return 0

    lax.fori_loop(0, N_CHUNKS, chunk, 0)


@jax.jit
def _run(seq_flat, token_table, pe_weight):
    mesh = plsc.VectorSubcoreMesh(core_axis_name="c", subcore_axis_name="s")
    return pl.kernel(
        _body,
        out_type=jax.ShapeDtypeStruct((BATCH * MAX_LEN, EMBED), jnp.float32),
        mesh=mesh,
        compiler_params=pltpu.CompilerParams(use_tc_tiling_on_sc=False),
        scratch_types=[
            pltpu.VMEM((MAX_LEN, EMBED), jnp.float32),
            pltpu.VMEM((ROWS,), jnp.int32),
            pltpu.VMEM((ROWS, EMBED), jnp.float32),
            pltpu.SemaphoreType.DMA,
            pltpu.SemaphoreType.DMA,
        ],
    )(seq_flat, token_table, pe_weight)


def kernel(sequence, token_table, pe_weight):
    seq_flat = sequence.reshape(-1).astype(jnp.int32)
    out = _run(seq_flat, token_table, pe_weight)
    return out.reshape(BATCH, MAX_LEN, EMBED)


# speedup vs baseline: 4.1495x; 1.1883x over previous
import jax
import jax.numpy as jnp
from jax import lax
from jax.experimental import pallas as pl
from jax.experimental.pallas import tpu as pltpu
from jax.experimental.pallas import tpu_sc as plsc

VOCAB = 100000
EMBED = 64
MAX_LEN = 200
BATCH = 4096

NUM_CORES = 2
NUM_SUBCORES = 16
NUM_WORKERS = NUM_CORES * NUM_SUBCORES
SEQ_PER_W = BATCH // NUM_WORKERS
ROWS = MAX_LEN
N_CHUNKS = SEQ_PER_W
N_QUADS = N_CHUNKS // 4
LANES = 16
COLS = EMBED // LANES


def _body(seq_hbm, table_hbm, pe_hbm, out_hbm, pe_v, idx_v,
          r0, r1, r2, r3, g0, g1, g2, g3, o0, o1, o2, o3):
    rows = (r0, r1, r2, r3)
    gsem = (g0, g1, g2, g3)
    osem = (o0, o1, o2, o3)
    wid = lax.axis_index("s") * NUM_CORES + lax.axis_index("c")
    base = wid * SEQ_PER_W * MAX_LEN

    pltpu.sync_copy(pe_hbm, pe_v)
    pltpu.sync_copy(seq_hbm.at[pl.ds(base, N_CHUNKS * ROWS)], idx_v)

    def gather_start(i, b):
        idx_slice = idx_v.at[pl.ds(i * ROWS, ROWS)]
        pltpu.make_async_copy(table_hbm.at[idx_slice], rows[b], gsem[b]).start()

    def gather_wait(i, b):
        idx_slice = idx_v.at[pl.ds(i * ROWS, ROWS)]
        pltpu.make_async_copy(table_hbm.at[idx_slice], rows[b], gsem[b]).wait()

    def out_start(i, b):
        dst = out_hbm.at[pl.ds(base + i * ROWS, ROWS)]
        pltpu.make_async_copy(rows[b], dst, osem[b]).start()

    def out_wait(i, b):
        dst = out_hbm.at[pl.ds(base + i * ROWS, ROWS)]
        pltpu.make_async_copy(rows[b], dst, osem[b]).wait()

    def add_pe(b):
        def add_row(r, _):
            for c in range(COLS):
                plsc.addupdate(
                    rows[b].at[r, pl.ds(c * LANES, LANES)],
                    pe_v[r, pl.ds(c * LANES, LANES)],
                )
            return 0
        lax.fori_loop(0, MAX_LEN, add_row, 0, unroll=8)

    gather_start(0, 0)
    gather_start(1, 1)

    def quad(q, _):
        i0 = q * 4
        for j in range(4):
            i = i0 + j
            b = j
            gather_wait(i, b)
            add_pe(b)
            out_start(i, b)
            nb = (j + 2) % 4
            if j < 2:
                @pl.when(q > 0)
                def _():
                    out_wait(i - 2, nb)
                gather_start(i + 2, nb)
            else:
                @pl.when(q < N_QUADS - 1)
                def _():
                    out_wait(i - 2, nb)
                    gather_start(i + 2, nb)
        return 0

    lax.fori_loop(0, N_QUADS, quad, 0)

    last = N_CHUNKS - 4
    for j in range(4):
        out_wait(last + j, j)


@jax.jit
def _run(seq_flat, token_table, pe_weight):
    mesh = plsc.VectorSubcoreMesh(core_axis_name="c", subcore_axis_name="s")
    return pl.kernel(
        _body,
        out_type=jax.ShapeDtypeStruct((BATCH * MAX_LEN, EMBED), jnp.float32),
        mesh=mesh,
        compiler_params=pltpu.CompilerParams(use_tc_tiling_on_sc=False),
        scratch_types=[
            pltpu.VMEM((MAX_LEN, EMBED), jnp.float32),
            pltpu.VMEM((N_CHUNKS * ROWS,), jnp.int32),
            pltpu.VMEM((ROWS, EMBED), jnp.float32),
            pltpu.VMEM((ROWS, EMBED), jnp.float32),
            pltpu.VMEM((ROWS, EMBED), jnp.float32),
            pltpu.VMEM((ROWS, EMBED), jnp.float32),
            pltpu.SemaphoreType.DMA,
            pltpu.SemaphoreType.DMA,
            pltpu.SemaphoreType.DMA,
            pltpu.SemaphoreType.DMA,
            pltpu.SemaphoreType.DMA,
            pltpu.SemaphoreType.DMA,
            pltpu.SemaphoreType.DMA,
            pltpu.SemaphoreType.DMA,
        ],
    )(seq_flat, token_table, pe_weight)


def kernel(sequence, token_table, pe_weight):
    seq_flat = sequence.reshape(-1).astype(jnp.int32)
    out = _run(seq_flat, token_table, pe_weight)
    return out.reshape(BATCH, MAX_LEN, EMBED)
